# baseline (speedup 1.0000x reference)
import jax
import jax.numpy as jnp
from jax import lax
from jax.experimental import pallas as pl
from jax.experimental.pallas import tpu as pltpu

R = 128
N = 32768
CB = 4096
NB = N // CB
NBUF = 8
SL = 8
NCH = R // SL


def _tc_argmax_body(x_hbm, o_ref, bufs, sems):
    def start(blk, slot):
        pltpu.make_async_copy(
            x_hbm.at[:, pl.ds(blk * CB, CB)], bufs.at[slot], sems.at[slot]
        ).start()

    def wait(slot):
        pltpu.make_async_copy(
            x_hbm.at[:, pl.ds(0, CB)], bufs.at[slot], sems.at[slot]
        ).wait()

    for j in range(NBUF):
        start(j, j)
    for b in range(NB):
        slot = b % NBUF
        wait(slot)
        buf = bufs.at[slot]
        vmax = buf[0:SL, :]
        vchunk = jnp.zeros((SL, CB), jnp.int32)
        for c in range(1, NCH):
            v = buf[SL * c:SL * (c + 1), :]
            p = v > vmax
            vmax = jnp.where(p, v, vmax)
            vchunk = jnp.where(p, jnp.int32(c), vchunk)
        m = jnp.max(vmax, axis=0)
        srow = lax.broadcasted_iota(jnp.int32, (SL, CB), 0)
        cand = jnp.where(vmax == m[None, :],
                         (vchunk << 3) | srow,
                         jnp.int32(R))
        o_ref[pl.ds(b * CB, CB)] = jnp.min(cand, axis=0)
        nxt = b + NBUF
        if nxt < NB:
            start(nxt, slot)


def kernel(x):
    out = pl.pallas_call(
        _tc_argmax_body,
        out_shape=jax.ShapeDtypeStruct((N,), jnp.int32),
        in_specs=[pl.BlockSpec(memory_space=pl.ANY)],
        out_specs=pl.BlockSpec((N,), lambda: (0,)),
        scratch_shapes=[
            pltpu.VMEM((NBUF, R, CB), jnp.float32),
            pltpu.SemaphoreType.DMA((NBUF,)),
        ],
    )(x)
    return out.astype(jnp.int64)


# speedup vs baseline: 1.0210x; 1.0210x over previous
import jax
import jax.numpy as jnp
from jax import lax
from jax.experimental import pallas as pl
from jax.experimental.pallas import tpu as pltpu

R = 128
N = 32768
CB = 2048
NB = N // CB
NBUF = 8
SL = 8
NCH = R // SL


def _tc_argmax_body(x_hbm, o_ref, bufs, sems):
    def start(blk, slot):
        pltpu.make_async_copy(
            x_hbm.at[:, pl.ds(blk * CB, CB)], bufs.at[slot], sems.at[slot]
        ).start()

    def wait(slot):
        pltpu.make_async_copy(
            x_hbm.at[:, pl.ds(0, CB)], bufs.at[slot], sems.at[slot]
        ).wait()

    for j in range(NBUF):
        start(j, j)
    for b in range(NB):
        slot = b % NBUF
        wait(slot)
        buf = bufs.at[slot]
        vmax = buf[0:SL, :]
        vchunk = jnp.zeros((SL, CB), jnp.int32)
        for c in range(1, NCH):
            v = buf[SL * c:SL * (c + 1), :]
            p = v > vmax
            vmax = jnp.where(p, v, vmax)
            vchunk = jnp.where(p, jnp.int32(c), vchunk)
        m = jnp.max(vmax, axis=0)
        srow = lax.broadcasted_iota(jnp.int32, (SL, CB), 0)
        cand = jnp.where(vmax == m[None, :],
                         (vchunk << 3) | srow,
                         jnp.int32(R))
        o_ref[pl.ds(b * CB, CB)] = jnp.min(cand, axis=0)
        nxt = b + NBUF
        if nxt < NB:
            start(nxt, slot)


def kernel(x):
    out = pl.pallas_call(
        _tc_argmax_body,
        out_shape=jax.ShapeDtypeStruct((N,), jnp.int32),
        in_specs=[pl.BlockSpec(memory_space=pl.ANY)],
        out_specs=pl.BlockSpec((N,), lambda: (0,)),
        scratch_shapes=[
            pltpu.VMEM((NBUF, R, CB), jnp.float32),
            pltpu.SemaphoreType.DMA((NBUF,)),
        ],
    )(x)
    return out.astype(jnp.int64)
